# baseline (speedup 1.0000x reference)
import functools

import jax
import jax.numpy as jnp
from jax import lax
from jax.experimental import pallas as pl
from jax.experimental.pallas import tpu as pltpu
from jax.experimental.pallas import tpu_sc as plsc

N = 10000
E = 320000
D = 128
LAMBDA1 = 0.5
LAMBDA2 = 0.5
GAMMA = 0.1

NC = 2
NS = 16
NW = NC * NS
EP = E // NW
C = 64
EPP = 10240
NCHUNK = EPP // C
BCH = 16
NBLK = NCHUNK // BCH
RB = 3
NP = 10240
ROWS_PER_TILE = NP // NS
ROW_CHUNK = C
NROWC = ROWS_PER_TILE // ROW_CHUNK


def _sc_mesh():
    return plsc.VectorSubcoreMesh(core_axis_name="c", subcore_axis_name="s")


def _deg_body(dst_hbm, out_hbm, dst_v, ones_v, zero_v, acc):
    cid = lax.axis_index("c")
    sid = lax.axis_index("s")

    ones = jnp.full((16,), 1.0, dtype=jnp.float32)
    zero = jnp.zeros((16,), dtype=jnp.float32)
    for i in range(8):
        ones_v[pl.ds(i * 16, 16)] = ones

    zchunk = NP // NS
    def zloop(i, _):
        zero_v[pl.ds(i * 16, 16)] = zero
        return 0
    lax.fori_loop(0, zchunk // 16, zloop, 0)
    pltpu.sync_copy(zero_v, acc.at[pl.ds(sid * zchunk, zchunk)])
    plsc.subcore_barrier()

    wid = cid * NS + sid
    pltpu.sync_copy(dst_hbm.at[wid], dst_v)

    def chunk(j, _):
        pltpu.sync_copy(ones_v.at[pl.ds(0, C)], acc.at[dst_v.at[j]], add=True)
        return 0
    lax.fori_loop(0, NCHUNK, chunk, 0)
    plsc.subcore_barrier()

    pltpu.sync_copy(acc.at[pl.ds(sid * zchunk, zchunk)], zero_v)
    pltpu.sync_copy(zero_v, out_hbm.at[cid, pl.ds(sid * zchunk, zchunk)])


def _deg_call(dst3):
    k = pl.kernel(
        _deg_body,
        out_type=jax.ShapeDtypeStruct((NC, NP), jnp.float32),
        mesh=_sc_mesh(),
        scratch_types=[
            pltpu.VMEM((NCHUNK, C), jnp.int32),
            pltpu.VMEM((128,), jnp.float32),
            pltpu.VMEM((NP // NS,), jnp.float32),
            pltpu.VMEM_SHARED((NP,), jnp.float32),
        ],
    )
    return k(dst3)


def _prop_body(y_hbm, src_hbm, dst_hbm, out_hbm,
               src_v, dst_v, rows0, rows1, rows2, acc,
               gsem0, gsem1, gsem2, ssem0, ssem1, ssem2):
    rows = (rows0, rows1, rows2)
    gsems = (gsem0, gsem1, gsem2)
    ssems = (ssem0, ssem1, ssem2)
    cid = lax.axis_index("c")
    sid = lax.axis_index("s")
    wid = cid * NS + sid

    zero = jnp.zeros((16,), dtype=jnp.float32)

    def zfill(i, _):
        for j in range(8):
            rows0[i, pl.ds(j * 16, 16)] = zero
        return 0
    lax.fori_loop(0, ROW_CHUNK, zfill, 0)
    r0 = sid * ROWS_PER_TILE
    for t in range(NROWC):
        pltpu.sync_copy(rows[0], acc.at[pl.ds(r0 + t * ROW_CHUNK, ROW_CHUNK), :])
    plsc.subcore_barrier()

    def gath(j, q):
        pltpu.async_copy(y_hbm.at[src_v.at[j]], rows[q], gsems[q])

    def wait_gath(q):
        pltpu.make_async_copy(y_hbm.at[src_v.at[0]], rows[q], gsems[q]).wait()

    def scat(j, q):
        pltpu.async_copy(rows[q], acc.at[dst_v.at[j]], ssems[q], add=True)

    def wait_scat(q):
        pltpu.make_async_copy(rows[q], acc.at[dst_v.at[0]], ssems[q]).wait()

    def block(b, _):
        pltpu.sync_copy(src_hbm.at[wid, pl.ds(b * BCH, BCH)], src_v)
        pltpu.sync_copy(dst_hbm.at[wid, pl.ds(b * BCH, BCH)], dst_v)

        gath(0, 0)
        gath(1, 1)
        for jb in range(BCH):
            q = jb % RB
            wait_gath(q)
            scat(jb, q)
            if jb + 2 < BCH:
                qq = (jb + 2) % RB
                if jb >= 1:
                    wait_scat(qq)
                gath(jb + 2, qq)
        for q in range(RB):
            wait_scat(q)
        return 0

    lax.fori_loop(0, NBLK, block, 0)
    plsc.subcore_barrier()

    for t in range(NROWC):
        rr = r0 + t * ROW_CHUNK
        pltpu.sync_copy(acc.at[pl.ds(rr, ROW_CHUNK), :], rows[0])
        pltpu.sync_copy(rows[0], out_hbm.at[cid, pl.ds(rr, ROW_CHUNK), :])


def _prop_call(y, src3, dst3):
    k = pl.kernel(
        _prop_body,
        out_type=jax.ShapeDtypeStruct((NC, NP, D), jnp.float32),
        mesh=_sc_mesh(),
        scratch_types=[
            pltpu.VMEM((BCH, C), jnp.int32),
            pltpu.VMEM((BCH, C), jnp.int32),
            pltpu.VMEM((C, D), jnp.float32),
            pltpu.VMEM((C, D), jnp.float32),
            pltpu.VMEM((C, D), jnp.float32),
            pltpu.VMEM_SHARED((NP, D), jnp.float32),
            pltpu.SemaphoreType.DMA,
            pltpu.SemaphoreType.DMA,
            pltpu.SemaphoreType.DMA,
            pltpu.SemaphoreType.DMA,
            pltpu.SemaphoreType.DMA,
            pltpu.SemaphoreType.DMA,
        ],
    )
    return k(y, src3, dst3)


BLK = 1000
GRID = N // BLK


def _t0_body(x_ref, d0_ref, d1_ref, w_ref, dis_ref, y_ref, h_ref, gram_ref):
    deg = d0_ref[...] + d1_ref[...]
    dis = jnp.where(deg > 0, lax.rsqrt(jnp.maximum(deg, 1.0)), 0.0)
    dis_ref[...] = dis
    x = x_ref[...]
    y_ref[...] = x * dis
    h = jnp.dot(x, w_ref[...], preferred_element_type=jnp.float32)
    h_ref[...] = h
    g = jnp.dot(h.T, h, preferred_element_type=jnp.float32)

    @pl.when(pl.program_id(0) == 0)
    def _():
        gram_ref[...] = jnp.zeros_like(gram_ref)
    gram_ref[...] += g


def _t0_call(x, d0, d1, w):
    return pl.pallas_call(
        _t0_body,
        grid=(GRID,),
        in_specs=[
            pl.BlockSpec((BLK, D), lambda i: (i, 0)),
            pl.BlockSpec((BLK, 1), lambda i: (i, 0)),
            pl.BlockSpec((BLK, 1), lambda i: (i, 0)),
            pl.BlockSpec((D, D), lambda i: (0, 0)),
        ],
        out_specs=[
            pl.BlockSpec((BLK, 1), lambda i: (i, 0)),
            pl.BlockSpec((BLK, D), lambda i: (i, 0)),
            pl.BlockSpec((BLK, D), lambda i: (i, 0)),
            pl.BlockSpec((D, D), lambda i: (0, 0)),
        ],
        out_shape=[
            jax.ShapeDtypeStruct((N, 1), jnp.float32),
            jax.ShapeDtypeStruct((N, D), jnp.float32),
            jax.ShapeDtypeStruct((N, D), jnp.float32),
            jax.ShapeDtypeStruct((D, D), jnp.float32),
        ],
    )(x, d0, d1, w)


def _combine_body(h_ref, gram_ref, p0_ref, p1_ref, dis_ref, w_ref, wn_ref,
                  y_ref, hn_ref, gramn_ref, *, last):
    h = h_ref[...]
    dis = dis_ref[...]
    p = (p0_ref[...] + p1_ref[...]) * dis
    agg = jnp.dot(p, w_ref[...], preferred_element_type=jnp.float32)
    hg = jnp.dot(h, gram_ref[...], preferred_element_type=jnp.float32)
    out = LAMBDA1 * agg + (LAMBDA2 + GAMMA) * h - GAMMA * hg
    if last:
        nrm = jnp.sqrt(jnp.sum(out * out, axis=1, keepdims=True))
        y_ref[...] = out / jnp.maximum(nrm, 1e-12)
    else:
        out = jnp.maximum(out, 0.0)
        y_ref[...] = out * dis
        hn = jnp.dot(out, wn_ref[...], preferred_element_type=jnp.float32)
        hn_ref[...] = hn
        g = jnp.dot(hn.T, hn, preferred_element_type=jnp.float32)

        @pl.when(pl.program_id(0) == 0)
        def _():
            gramn_ref[...] = jnp.zeros_like(gramn_ref)
        gramn_ref[...] += g


def _combine_call(h, gram, p0, p1, dis, w, wn, last):
    blk2 = lambda i: (i, 0)
    in_specs = [
        pl.BlockSpec((BLK, D), blk2),
        pl.BlockSpec((D, D), lambda i: (0, 0)),
        pl.BlockSpec((BLK, D), blk2),
        pl.BlockSpec((BLK, D), blk2),
        pl.BlockSpec((BLK, 1), blk2),
        pl.BlockSpec((D, D), lambda i: (0, 0)),
        pl.BlockSpec((D, D), lambda i: (0, 0)),
    ]
    if last:
        out_specs = [pl.BlockSpec((BLK, D), blk2)]
        out_shape = [jax.ShapeDtypeStruct((N, D), jnp.float32)]
    else:
        out_specs = [
            pl.BlockSpec((BLK, D), blk2),
            pl.BlockSpec((BLK, D), blk2),
            pl.BlockSpec((D, D), lambda i: (0, 0)),
        ]
        out_shape = [
            jax.ShapeDtypeStruct((N, D), jnp.float32),
            jax.ShapeDtypeStruct((N, D), jnp.float32),
            jax.ShapeDtypeStruct((D, D), jnp.float32),
        ]
    body = functools.partial(_combine_body, last=last)

    def wrapped(*refs):
        if last:
            h_r, g_r, p0_r, p1_r, dis_r, w_r, wn_r, y_r = refs
            body(h_r, g_r, p0_r, p1_r, dis_r, w_r, wn_r, y_r, None, None)
        else:
            body(*refs)

    return pl.pallas_call(
        wrapped,
        grid=(GRID,),
        in_specs=in_specs,
        out_specs=out_specs,
        out_shape=out_shape,
    )(h, gram, p0, p1, dis, w, wn)


def kernel(x, edge_index, W_in, W_mid0, W_mid1, W_out):
    src_u = edge_index[0].astype(jnp.int32)
    dst_u = edge_index[1].astype(jnp.int32)
    order = jnp.argsort(src_u)
    srcw = src_u[order].reshape(NW, EP)
    dstw = dst_u[order].reshape(NW, EP)
    src3 = jnp.pad(srcw, ((0, 0), (0, EPP - EP))).reshape(NW, NCHUNK, C)
    dst3 = jnp.pad(dstw, ((0, 0), (0, EPP - EP)),
                   constant_values=NP - 1).reshape(NW, NCHUNK, C)

    degp = _deg_call(dst3)
    d0 = degp[0, :N].reshape(N, 1)
    d1 = degp[1, :N].reshape(N, 1)

    dis, y, h, gram = _t0_call(x, d0, d1, W_in)

    ws = (W_in, W_mid0, W_mid1, W_out)
    for l in range(4):
        pp = _prop_call(y, src3, dst3)
        last = l == 3
        wn = ws[l + 1] if not last else ws[l]
        res = _combine_call(h, gram, pp[0], pp[1], dis, ws[l], wn, last)
        if last:
            return res[0]
        y, h, gram = res


# speedup vs baseline: 1.5739x; 1.5739x over previous
import functools

import jax
import jax.numpy as jnp
from jax import lax
from jax.experimental import pallas as pl
from jax.experimental.pallas import tpu as pltpu
from jax.experimental.pallas import tpu_sc as plsc

N = 10000
E = 320000
D = 128
LAMBDA1 = 0.5
LAMBDA2 = 0.5
GAMMA = 0.1

NC = 2
NS = 16
NW = NC * NS
EP = E // NW
C = 64
EPP = 10240
NCHUNK = EPP // C
BCH = 32
NBLK = NCHUNK // BCH
RB = 3
NP = 10240
ROWS_PER_TILE = NP // NS
ROW_CHUNK = C
NROWC = ROWS_PER_TILE // ROW_CHUNK


def _sc_mesh():
    return plsc.VectorSubcoreMesh(core_axis_name="c", subcore_axis_name="s")


def _deg_body(dst_hbm, out_hbm, dst_v, ones_v, zero_v, acc):
    cid = lax.axis_index("c")
    sid = lax.axis_index("s")

    ones = jnp.full((16,), 1.0, dtype=jnp.float32)
    zero = jnp.zeros((16,), dtype=jnp.float32)
    for i in range(8):
        ones_v[pl.ds(i * 16, 16)] = ones

    zchunk = NP // NS
    def zloop(i, _):
        zero_v[pl.ds(i * 16, 16)] = zero
        return 0
    lax.fori_loop(0, zchunk // 16, zloop, 0)
    pltpu.sync_copy(zero_v, acc.at[pl.ds(sid * zchunk, zchunk)])
    plsc.subcore_barrier()

    wid = cid * NS + sid
    pltpu.sync_copy(dst_hbm.at[wid], dst_v)

    def chunk(j, _):
        pltpu.sync_copy(ones_v.at[pl.ds(0, C)], acc.at[dst_v.at[j]], add=True)
        return 0
    lax.fori_loop(0, NCHUNK, chunk, 0)
    plsc.subcore_barrier()

    pltpu.sync_copy(acc.at[pl.ds(sid * zchunk, zchunk)], zero_v)
    pltpu.sync_copy(zero_v, out_hbm.at[cid, pl.ds(sid * zchunk, zchunk)])


def _deg_call(dst3):
    k = pl.kernel(
        _deg_body,
        out_type=jax.ShapeDtypeStruct((NC, NP), jnp.float32),
        mesh=_sc_mesh(),
        scratch_types=[
            pltpu.VMEM((NCHUNK, C), jnp.int32),
            pltpu.VMEM((128,), jnp.float32),
            pltpu.VMEM((NP // NS,), jnp.float32),
            pltpu.VMEM_SHARED((NP,), jnp.float32),
        ],
    )
    return k(dst3)


def _prop_body(y_hbm, src_hbm, dst_hbm, out_hbm,
               src_v, dst_v, rows0, rows1, rows2, acc,
               gsem0, gsem1, gsem2, ssem0, ssem1, ssem2):
    rows = (rows0, rows1, rows2)
    gsems = (gsem0, gsem1, gsem2)
    ssems = (ssem0, ssem1, ssem2)
    cid = lax.axis_index("c")
    sid = lax.axis_index("s")
    wid = cid * NS + sid

    zero = jnp.zeros((16,), dtype=jnp.float32)

    def zfill(i, _):
        for j in range(8):
            rows0[i, pl.ds(j * 16, 16)] = zero
        return 0
    lax.fori_loop(0, ROW_CHUNK, zfill, 0)
    r0 = sid * ROWS_PER_TILE
    for t in range(NROWC):
        pltpu.sync_copy(rows[0], acc.at[pl.ds(r0 + t * ROW_CHUNK, ROW_CHUNK), :])
    plsc.subcore_barrier()

    def gath(j, q):
        pltpu.async_copy(y_hbm.at[src_v.at[j]], rows[q], gsems[q])

    def wait_gath(q):
        pltpu.make_async_copy(y_hbm.at[src_v.at[0]], rows[q], gsems[q]).wait()

    def scat(j, q):
        pltpu.async_copy(rows[q], acc.at[dst_v.at[j]], ssems[q], add=True)

    def wait_scat(q):
        pltpu.make_async_copy(rows[q], acc.at[dst_v.at[0]], ssems[q]).wait()

    def block(b, _):
        pltpu.sync_copy(src_hbm.at[wid, pl.ds(b * BCH, BCH)], src_v)
        pltpu.sync_copy(dst_hbm.at[wid, pl.ds(b * BCH, BCH)], dst_v)

        gath(0, 0)
        gath(1, 1)
        for jb in range(BCH):
            q = jb % RB
            wait_gath(q)
            scat(jb, q)
            if jb + 2 < BCH:
                qq = (jb + 2) % RB
                if jb >= 1:
                    wait_scat(qq)
                gath(jb + 2, qq)
        for q in range(RB):
            wait_scat(q)
        return 0

    lax.fori_loop(0, NBLK, block, 0)
    plsc.subcore_barrier()

    for t in range(NROWC):
        rr = r0 + t * ROW_CHUNK
        pltpu.sync_copy(acc.at[pl.ds(rr, ROW_CHUNK), :], rows[0])
        pltpu.sync_copy(rows[0], out_hbm.at[cid, pl.ds(rr, ROW_CHUNK), :])


def _prop_call(y, src3, dst3):
    k = pl.kernel(
        _prop_body,
        out_type=jax.ShapeDtypeStruct((NC, NP, D), jnp.float32),
        mesh=_sc_mesh(),
        scratch_types=[
            pltpu.VMEM((BCH, C), jnp.int32),
            pltpu.VMEM((BCH, C), jnp.int32),
            pltpu.VMEM((C, D), jnp.float32),
            pltpu.VMEM((C, D), jnp.float32),
            pltpu.VMEM((C, D), jnp.float32),
            pltpu.VMEM_SHARED((NP, D), jnp.float32),
            pltpu.SemaphoreType.DMA,
            pltpu.SemaphoreType.DMA,
            pltpu.SemaphoreType.DMA,
            pltpu.SemaphoreType.DMA,
            pltpu.SemaphoreType.DMA,
            pltpu.SemaphoreType.DMA,
        ],
    )
    return k(y, src3, dst3)


BLK = 1000
GRID = N // BLK


def _t0_body(x_ref, d0_ref, d1_ref, w_ref, dis_ref, y_ref, h_ref, gram_ref):
    deg = d0_ref[...] + d1_ref[...]
    dis = jnp.where(deg > 0, lax.rsqrt(jnp.maximum(deg, 1.0)), 0.0)
    dis_ref[...] = dis
    x = x_ref[...]
    y_ref[...] = x * dis
    h = jnp.dot(x, w_ref[...], preferred_element_type=jnp.float32)
    h_ref[...] = h
    g = jnp.dot(h.T, h, preferred_element_type=jnp.float32)

    @pl.when(pl.program_id(0) == 0)
    def _():
        gram_ref[...] = jnp.zeros_like(gram_ref)
    gram_ref[...] += g


def _t0_call(x, d0, d1, w):
    return pl.pallas_call(
        _t0_body,
        grid=(GRID,),
        in_specs=[
            pl.BlockSpec((BLK, D), lambda i: (i, 0)),
            pl.BlockSpec((BLK, 1), lambda i: (i, 0)),
            pl.BlockSpec((BLK, 1), lambda i: (i, 0)),
            pl.BlockSpec((D, D), lambda i: (0, 0)),
        ],
        out_specs=[
            pl.BlockSpec((BLK, 1), lambda i: (i, 0)),
            pl.BlockSpec((BLK, D), lambda i: (i, 0)),
            pl.BlockSpec((BLK, D), lambda i: (i, 0)),
            pl.BlockSpec((D, D), lambda i: (0, 0)),
        ],
        out_shape=[
            jax.ShapeDtypeStruct((N, 1), jnp.float32),
            jax.ShapeDtypeStruct((N, D), jnp.float32),
            jax.ShapeDtypeStruct((N, D), jnp.float32),
            jax.ShapeDtypeStruct((D, D), jnp.float32),
        ],
    )(x, d0, d1, w)


def _combine_body(h_ref, gram_ref, p0_ref, p1_ref, dis_ref, w_ref, wn_ref,
                  y_ref, hn_ref, gramn_ref, *, last):
    h = h_ref[...]
    dis = dis_ref[...]
    p = (p0_ref[...] + p1_ref[...]) * dis
    agg = jnp.dot(p, w_ref[...], preferred_element_type=jnp.float32)
    hg = jnp.dot(h, gram_ref[...], preferred_element_type=jnp.float32)
    out = LAMBDA1 * agg + (LAMBDA2 + GAMMA) * h - GAMMA * hg
    if last:
        nrm = jnp.sqrt(jnp.sum(out * out, axis=1, keepdims=True))
        y_ref[...] = out / jnp.maximum(nrm, 1e-12)
    else:
        out = jnp.maximum(out, 0.0)
        y_ref[...] = out * dis
        hn = jnp.dot(out, wn_ref[...], preferred_element_type=jnp.float32)
        hn_ref[...] = hn
        g = jnp.dot(hn.T, hn, preferred_element_type=jnp.float32)

        @pl.when(pl.program_id(0) == 0)
        def _():
            gramn_ref[...] = jnp.zeros_like(gramn_ref)
        gramn_ref[...] += g


def _combine_call(h, gram, p0, p1, dis, w, wn, last):
    blk2 = lambda i: (i, 0)
    in_specs = [
        pl.BlockSpec((BLK, D), blk2),
        pl.BlockSpec((D, D), lambda i: (0, 0)),
        pl.BlockSpec((BLK, D), blk2),
        pl.BlockSpec((BLK, D), blk2),
        pl.BlockSpec((BLK, 1), blk2),
        pl.BlockSpec((D, D), lambda i: (0, 0)),
        pl.BlockSpec((D, D), lambda i: (0, 0)),
    ]
    if last:
        out_specs = [pl.BlockSpec((BLK, D), blk2)]
        out_shape = [jax.ShapeDtypeStruct((N, D), jnp.float32)]
    else:
        out_specs = [
            pl.BlockSpec((BLK, D), blk2),
            pl.BlockSpec((BLK, D), blk2),
            pl.BlockSpec((D, D), lambda i: (0, 0)),
        ]
        out_shape = [
            jax.ShapeDtypeStruct((N, D), jnp.float32),
            jax.ShapeDtypeStruct((N, D), jnp.float32),
            jax.ShapeDtypeStruct((D, D), jnp.float32),
        ]
    body = functools.partial(_combine_body, last=last)

    def wrapped(*refs):
        if last:
            h_r, g_r, p0_r, p1_r, dis_r, w_r, wn_r, y_r = refs
            body(h_r, g_r, p0_r, p1_r, dis_r, w_r, wn_r, y_r, None, None)
        else:
            body(*refs)

    return pl.pallas_call(
        wrapped,
        grid=(GRID,),
        in_specs=in_specs,
        out_specs=out_specs,
        out_shape=out_shape,
    )(h, gram, p0, p1, dis, w, wn)


def kernel(x, edge_index, W_in, W_mid0, W_mid1, W_out):
    srcw = edge_index[0].astype(jnp.int32).reshape(NW, EP)
    dstw = edge_index[1].astype(jnp.int32).reshape(NW, EP)
    src3 = jnp.pad(srcw, ((0, 0), (0, EPP - EP))).reshape(NW, NCHUNK, C)
    dst3 = jnp.pad(dstw, ((0, 0), (0, EPP - EP)),
                   constant_values=NP - 1).reshape(NW, NCHUNK, C)

    degp = _deg_call(dst3)
    d0 = degp[0, :N].reshape(N, 1)
    d1 = degp[1, :N].reshape(N, 1)

    dis, y, h, gram = _t0_call(x, d0, d1, W_in)

    ws = (W_in, W_mid0, W_mid1, W_out)
    for l in range(4):
        pp = _prop_call(y, src3, dst3)
        last = l == 3
        wn = ws[l + 1] if not last else ws[l]
        res = _combine_call(h, gram, pp[0], pp[1], dis, ws[l], wn, last)
        if last:
            return res[0]
        y, h, gram = res


# speedup vs baseline: 1.6035x; 1.0188x over previous
import functools

import jax
import jax.numpy as jnp
from jax import lax
from jax.experimental import pallas as pl
from jax.experimental.pallas import tpu as pltpu
from jax.experimental.pallas import tpu_sc as plsc

N = 10000
E = 320000
D = 128
LAMBDA1 = 0.5
LAMBDA2 = 0.5
GAMMA = 0.1

NC = 2
NS = 16
NW = NC * NS
EP = E // NW
C = 80
EPP = 10240
NCHUNK = EPP // C
BCH = 32
NBLK = NCHUNK // BCH
RB = 3
NP = 10240
ROWS_PER_TILE = NP // NS
ROW_CHUNK = C
NROWC = ROWS_PER_TILE // ROW_CHUNK


def _sc_mesh():
    return plsc.VectorSubcoreMesh(core_axis_name="c", subcore_axis_name="s")


def _deg_body(dst_hbm, out_hbm, dst_v, ones_v, zero_v, acc):
    cid = lax.axis_index("c")
    sid = lax.axis_index("s")

    ones = jnp.full((16,), 1.0, dtype=jnp.float32)
    zero = jnp.zeros((16,), dtype=jnp.float32)
    for i in range(8):
        ones_v[pl.ds(i * 16, 16)] = ones

    zchunk = NP // NS
    def zloop(i, _):
        zero_v[pl.ds(i * 16, 16)] = zero
        return 0
    lax.fori_loop(0, zchunk // 16, zloop, 0)
    pltpu.sync_copy(zero_v, acc.at[pl.ds(sid * zchunk, zchunk)])
    plsc.subcore_barrier()

    wid = cid * NS + sid
    pltpu.sync_copy(dst_hbm.at[wid], dst_v)

    def chunk(j, _):
        pltpu.sync_copy(ones_v.at[pl.ds(0, C)], acc.at[dst_v.at[j]], add=True)
        return 0
    lax.fori_loop(0, NCHUNK, chunk, 0)
    plsc.subcore_barrier()

    pltpu.sync_copy(acc.at[pl.ds(sid * zchunk, zchunk)], zero_v)
    pltpu.sync_copy(zero_v, out_hbm.at[cid, pl.ds(sid * zchunk, zchunk)])


def _deg_call(dst3):
    k = pl.kernel(
        _deg_body,
        out_type=jax.ShapeDtypeStruct((NC, NP), jnp.float32),
        mesh=_sc_mesh(),
        scratch_types=[
            pltpu.VMEM((NCHUNK, C), jnp.int32),
            pltpu.VMEM((128,), jnp.float32),
            pltpu.VMEM((NP // NS,), jnp.float32),
            pltpu.VMEM_SHARED((NP,), jnp.float32),
        ],
    )
    return k(dst3)


def _prop_body(y_hbm, src_hbm, dst_hbm, out_hbm,
               src_v, dst_v, rows0, rows1, rows2, acc,
               gsem0, gsem1, gsem2, ssem0, ssem1, ssem2):
    rows = (rows0, rows1, rows2)
    gsems = (gsem0, gsem1, gsem2)
    ssems = (ssem0, ssem1, ssem2)
    cid = lax.axis_index("c")
    sid = lax.axis_index("s")
    wid = cid * NS + sid

    zero = jnp.zeros((16,), dtype=jnp.float32)

    def zfill(i, _):
        for j in range(8):
            rows0[i, pl.ds(j * 16, 16)] = zero
        return 0
    lax.fori_loop(0, ROW_CHUNK, zfill, 0)
    r0 = sid * ROWS_PER_TILE
    for t in range(NROWC):
        pltpu.sync_copy(rows[0], acc.at[pl.ds(r0 + t * ROW_CHUNK, ROW_CHUNK), :])
    plsc.subcore_barrier()

    def gath(j, q):
        pltpu.async_copy(y_hbm.at[src_v.at[j]], rows[q], gsems[q])

    def wait_gath(q):
        pltpu.make_async_copy(y_hbm.at[src_v.at[0]], rows[q], gsems[q]).wait()

    def scat(j, q):
        pltpu.async_copy(rows[q], acc.at[dst_v.at[j]], ssems[q], add=True)

    def wait_scat(q):
        pltpu.make_async_copy(rows[q], acc.at[dst_v.at[0]], ssems[q]).wait()

    def block(b, _):
        pltpu.sync_copy(src_hbm.at[wid, pl.ds(b * BCH, BCH)], src_v)
        pltpu.sync_copy(dst_hbm.at[wid, pl.ds(b * BCH, BCH)], dst_v)

        gath(0, 0)
        gath(1, 1)
        for jb in range(BCH):
            q = jb % RB
            wait_gath(q)
            scat(jb, q)
            if jb + 2 < BCH:
                qq = (jb + 2) % RB
                if jb >= 1:
                    wait_scat(qq)
                gath(jb + 2, qq)
        for q in range(RB):
            wait_scat(q)
        return 0

    lax.fori_loop(0, NBLK, block, 0)
    plsc.subcore_barrier()

    for t in range(NROWC):
        rr = r0 + t * ROW_CHUNK
        pltpu.sync_copy(acc.at[pl.ds(rr, ROW_CHUNK), :], rows[0])
        pltpu.sync_copy(rows[0], out_hbm.at[cid, pl.ds(rr, ROW_CHUNK), :])


def _prop_call(y, src3, dst3):
    k = pl.kernel(
        _prop_body,
        out_type=jax.ShapeDtypeStruct((NC, NP, D), jnp.float32),
        mesh=_sc_mesh(),
        scratch_types=[
            pltpu.VMEM((BCH, C), jnp.int32),
            pltpu.VMEM((BCH, C), jnp.int32),
            pltpu.VMEM((C, D), jnp.float32),
            pltpu.VMEM((C, D), jnp.float32),
            pltpu.VMEM((C, D), jnp.float32),
            pltpu.VMEM_SHARED((NP, D), jnp.float32),
            pltpu.SemaphoreType.DMA,
            pltpu.SemaphoreType.DMA,
            pltpu.SemaphoreType.DMA,
            pltpu.SemaphoreType.DMA,
            pltpu.SemaphoreType.DMA,
            pltpu.SemaphoreType.DMA,
        ],
    )
    return k(y, src3, dst3)


BLK = 1000
GRID = N // BLK


def _t0_body(x_ref, d0_ref, d1_ref, w_ref, dis_ref, y_ref, h_ref, gram_ref):
    deg = d0_ref[...] + d1_ref[...]
    dis = jnp.where(deg > 0, lax.rsqrt(jnp.maximum(deg, 1.0)), 0.0)
    dis_ref[...] = dis
    x = x_ref[...]
    y_ref[...] = x * dis
    h = jnp.dot(x, w_ref[...], preferred_element_type=jnp.float32)
    h_ref[...] = h
    g = jnp.dot(h.T, h, preferred_element_type=jnp.float32)

    @pl.when(pl.program_id(0) == 0)
    def _():
        gram_ref[...] = jnp.zeros_like(gram_ref)
    gram_ref[...] += g


def _t0_call(x, d0, d1, w):
    return pl.pallas_call(
        _t0_body,
        grid=(GRID,),
        in_specs=[
            pl.BlockSpec((BLK, D), lambda i: (i, 0)),
            pl.BlockSpec((BLK, 1), lambda i: (i, 0)),
            pl.BlockSpec((BLK, 1), lambda i: (i, 0)),
            pl.BlockSpec((D, D), lambda i: (0, 0)),
        ],
        out_specs=[
            pl.BlockSpec((BLK, 1), lambda i: (i, 0)),
            pl.BlockSpec((BLK, D), lambda i: (i, 0)),
            pl.BlockSpec((BLK, D), lambda i: (i, 0)),
            pl.BlockSpec((D, D), lambda i: (0, 0)),
        ],
        out_shape=[
            jax.ShapeDtypeStruct((N, 1), jnp.float32),
            jax.ShapeDtypeStruct((N, D), jnp.float32),
            jax.ShapeDtypeStruct((N, D), jnp.float32),
            jax.ShapeDtypeStruct((D, D), jnp.float32),
        ],
    )(x, d0, d1, w)


def _combine_body(h_ref, gram_ref, p0_ref, p1_ref, dis_ref, w_ref, wn_ref,
                  y_ref, hn_ref, gramn_ref, *, last):
    h = h_ref[...]
    dis = dis_ref[...]
    p = (p0_ref[...] + p1_ref[...]) * dis
    agg = jnp.dot(p, w_ref[...], preferred_element_type=jnp.float32)
    hg = jnp.dot(h, gram_ref[...], preferred_element_type=jnp.float32)
    out = LAMBDA1 * agg + (LAMBDA2 + GAMMA) * h - GAMMA * hg
    if last:
        nrm = jnp.sqrt(jnp.sum(out * out, axis=1, keepdims=True))
        y_ref[...] = out / jnp.maximum(nrm, 1e-12)
    else:
        out = jnp.maximum(out, 0.0)
        y_ref[...] = out * dis
        hn = jnp.dot(out, wn_ref[...], preferred_element_type=jnp.float32)
        hn_ref[...] = hn
        g = jnp.dot(hn.T, hn, preferred_element_type=jnp.float32)

        @pl.when(pl.program_id(0) == 0)
        def _():
            gramn_ref[...] = jnp.zeros_like(gramn_ref)
        gramn_ref[...] += g


def _combine_call(h, gram, p0, p1, dis, w, wn, last):
    blk2 = lambda i: (i, 0)
    in_specs = [
        pl.BlockSpec((BLK, D), blk2),
        pl.BlockSpec((D, D), lambda i: (0, 0)),
        pl.BlockSpec((BLK, D), blk2),
        pl.BlockSpec((BLK, D), blk2),
        pl.BlockSpec((BLK, 1), blk2),
        pl.BlockSpec((D, D), lambda i: (0, 0)),
        pl.BlockSpec((D, D), lambda i: (0, 0)),
    ]
    if last:
        out_specs = [pl.BlockSpec((BLK, D), blk2)]
        out_shape = [jax.ShapeDtypeStruct((N, D), jnp.float32)]
    else:
        out_specs = [
            pl.BlockSpec((BLK, D), blk2),
            pl.BlockSpec((BLK, D), blk2),
            pl.BlockSpec((D, D), lambda i: (0, 0)),
        ]
        out_shape = [
            jax.ShapeDtypeStruct((N, D), jnp.float32),
            jax.ShapeDtypeStruct((N, D), jnp.float32),
            jax.ShapeDtypeStruct((D, D), jnp.float32),
        ]
    body = functools.partial(_combine_body, last=last)

    def wrapped(*refs):
        if last:
            h_r, g_r, p0_r, p1_r, dis_r, w_r, wn_r, y_r = refs
            body(h_r, g_r, p0_r, p1_r, dis_r, w_r, wn_r, y_r, None, None)
        else:
            body(*refs)

    return pl.pallas_call(
        wrapped,
        grid=(GRID,),
        in_specs=in_specs,
        out_specs=out_specs,
        out_shape=out_shape,
    )(h, gram, p0, p1, dis, w, wn)


def kernel(x, edge_index, W_in, W_mid0, W_mid1, W_out):
    srcw = edge_index[0].astype(jnp.int32).reshape(NW, EP)
    dstw = edge_index[1].astype(jnp.int32).reshape(NW, EP)
    src3 = jnp.pad(srcw, ((0, 0), (0, EPP - EP))).reshape(NW, NCHUNK, C)
    dst3 = jnp.pad(dstw, ((0, 0), (0, EPP - EP)),
                   constant_values=NP - 1).reshape(NW, NCHUNK, C)

    degp = _deg_call(dst3)
    d0 = degp[0, :N].reshape(N, 1)
    d1 = degp[1, :N].reshape(N, 1)

    dis, y, h, gram = _t0_call(x, d0, d1, W_in)

    ws = (W_in, W_mid0, W_mid1, W_out)
    for l in range(4):
        pp = _prop_call(y, src3, dst3)
        last = l == 3
        wn = ws[l + 1] if not last else ws[l]
        res = _combine_call(h, gram, pp[0], pp[1], dis, ws[l], wn, last)
        if last:
            return res[0]
        y, h, gram = res
